# baseline (speedup 1.0000x reference)
import functools

import jax
import jax.numpy as jnp
from jax.experimental import pallas as pl
from jax.experimental.pallas import tpu as pltpu

_NHEADS = 16
_TS = 256
_TQ = 512
_TK = 512

_INT_MIN_VAL = -2147483648


def _layer_norm(x, scale, bias, eps=1e-06):
    mean = jnp.mean(x, axis=-1, keepdims=True)
    var = jnp.mean(jnp.square(x - mean), axis=-1, keepdims=True)
    return (x - mean) / jnp.sqrt(var + eps) * scale + bias


def _softmax_colsum(s):
    m = jnp.max(s, axis=-1, keepdims=True)
    e = jnp.exp(s - m)
    p = e * (1.0 / jnp.sum(e, axis=-1, keepdims=True))
    return jnp.sum(p, axis=0, keepdims=True)


def _gate_dense(scores, tau, k):
    int_min = jnp.int32(_INT_MIN_VAL)
    i32 = jax.lax.bitcast_convert_type(scores, jnp.int32)
    u = jnp.where(i32 < 0, ~i32, i32 | int_min)
    su = u ^ int_min

    rows = scores.shape[0]
    lo0 = jnp.min(su, axis=-1, keepdims=True)
    hi0 = jnp.max(su, axis=-1, keepdims=True)
    gap0 = hi0 - lo0
    done0 = ((gap0 == 0) | (gap0 == 1)).astype(jnp.int32)
    thr0 = lo0
    kf = jnp.float32(k)

    def cond(c):
        it, ndone = c[0], c[1]
        return jnp.logical_and(it < 80, ndone < rows)

    def body(c):
        it, ndone, lo, hi, thr, done, clo, chi = c
        hgap = jax.lax.shift_right_logical(hi - lo, 1)
        qgap = jnp.maximum(jax.lax.shift_right_logical(hi - lo, 2), 1)
        frac = jnp.clip((clo - kf) / jnp.maximum(clo - chi, 1.0), 0.25, 0.75)
        off = (hgap.astype(jnp.float32) * (2.0 * frac)).astype(jnp.int32)
        mid = jnp.clip(lo + off, lo + qgap, hi - qgap)
        cnt = jnp.sum((su >= mid).astype(jnp.int32), axis=-1, keepdims=True)
        cntf = cnt.astype(jnp.float32)
        ge = cnt >= k
        lo2 = jnp.where(ge, mid, lo)
        hi2 = jnp.where(ge, hi, mid)
        clo2 = jnp.where(ge, cntf, clo)
        chi2 = jnp.where(ge, chi, cntf)
        hit = cnt == k
        gap = hi2 - lo2
        collapse = (gap == 0) | (gap == 1)
        fresh = done == 0
        thr2 = jnp.where(fresh, jnp.where(hit, mid, lo2), thr)
        done2 = jnp.where(fresh & (hit | collapse), 1, done)
        return (it + 1, jnp.sum(done2),
                jnp.where(fresh, lo2, lo), jnp.where(fresh, hi2, hi),
                thr2, done2,
                jnp.where(fresh, clo2, clo), jnp.where(fresh, chi2, chi))

    init = (jnp.int32(0), jnp.sum(done0), lo0, hi0, thr0, done0,
            jnp.full((rows, 1), float(scores.shape[-1]), jnp.float32),
            jnp.ones((rows, 1), jnp.float32))
    thr = jax.lax.while_loop(cond, body, init)[4]
    mask = su >= thr

    raw = scores - tau
    meg = jnp.where(mask & (raw > 0), jnp.exp(raw) - 1.0, 0.0)
    gate_sum = jnp.sum(meg, axis=-1, keepdims=True) + 1e-08
    strength = jnp.tanh(jnp.max(meg, axis=-1, keepdims=True))
    return meg * (strength / gate_sum)


def _bottleneck(gate, enc, dec):
    h = jnp.dot(gate.astype(jnp.bfloat16), enc.astype(jnp.bfloat16),
                preferred_element_type=jnp.float32)
    return jnp.dot(h.astype(jnp.bfloat16), dec.astype(jnp.bfloat16),
                   preferred_element_type=jnp.float32)


def _inv_norms(embT):
    return 1.0 / (jnp.sqrt(jnp.sum(embT * embT, axis=0, keepdims=True)) + 1e-08)


def _attn_prep_kernel(x_ref, ln1s_ref, ln1b_ref, wcat_ref, bcat_ref,
                      qkT_ref, vT_ref, qk_enc_ref, qk_dec_ref,
                      v_enc_ref, v_dec_ref,
                      q_out, k_out, v_out, uq_out, uv_out, *, kqk, kv):
    i = pl.program_id(0)
    h1 = _layer_norm(x_ref[...], ln1s_ref[...], ln1b_ref[...])
    hp = jnp.dot(h1, wcat_ref[...], preferred_element_type=jnp.float32)
    hp = hp + bcat_ref[...]
    h_q, h_k, h_v = hp[:, 0:64], hp[:, 64:128], hp[:, 128:192]
    tau_q, tau_k, tau_v = hp[:, 192:193], hp[:, 193:194], hp[:, 194:195]

    qkT = qkT_ref[...]
    inv_qk = _inv_norms(qkT)
    s_q = jnp.dot(h_q, qkT, preferred_element_type=jnp.float32) * inv_qk
    s_k = jnp.dot(h_k, qkT, preferred_element_type=jnp.float32) * inv_qk
    vT = vT_ref[...]
    s_v = jnp.dot(h_v, vT, preferred_element_type=jnp.float32) * _inv_norms(vT)

    ts = s_q.shape[0]
    s_all = jnp.concatenate([s_q, s_k, s_v], axis=0)
    tau_all = jnp.concatenate([tau_q, tau_k, tau_v], axis=0)
    g_all = _gate_dense(s_all, tau_all, kqk)
    qk_enc, qk_dec = qk_enc_ref[...], qk_dec_ref[...]
    qk_both = _bottleneck(g_all[0:2 * ts], qk_enc, qk_dec)
    q_out[...] = qk_both[0:ts]
    k_out[...] = qk_both[ts:2 * ts]
    v_out[...] = _bottleneck(g_all[2 * ts:3 * ts],
                             v_enc_ref[...], v_dec_ref[...])

    @pl.when(i == 0)
    def _():
        uq_out[...] = jnp.zeros_like(uq_out)
        uv_out[...] = jnp.zeros_like(uv_out)

    uq_out[...] += _softmax_colsum(s_q)
    uv_out[...] += _softmax_colsum(s_v)


def _attn_kernel(q_ref, k_ref, v_ref, o_ref, m_scr, l_scr, acc_scr):
    qi, kj = pl.program_id(1), pl.program_id(2)

    @pl.when(kj == 0)
    def _():
        m_scr[...] = jnp.full_like(m_scr, -1e30)
        l_scr[...] = jnp.zeros_like(l_scr)
        acc_scr[...] = jnp.zeros_like(acc_scr)

    @pl.when(kj <= qi)
    def _():
        q = (q_ref[0] * 0.125).astype(jnp.bfloat16)
        s = jax.lax.dot_general(q, k_ref[0].astype(jnp.bfloat16),
                                (((1,), (1,)), ((), ())),
                                preferred_element_type=jnp.float32)
        row = qi * _TQ + jax.lax.broadcasted_iota(jnp.int32, s.shape, 0)
        col = kj * _TK + jax.lax.broadcasted_iota(jnp.int32, s.shape, 1)
        s = jnp.where(col <= row, s, -1e30)
        m_old = m_scr[...]
        m_new = jnp.maximum(m_old, jnp.max(s, axis=-1, keepdims=True))
        alpha = jnp.exp(m_old - m_new)
        p = jnp.exp(s - m_new)
        l_scr[...] = l_scr[...] * alpha + jnp.sum(p, axis=-1, keepdims=True)
        acc_scr[...] = acc_scr[...] * alpha + jnp.dot(
            p.astype(jnp.bfloat16), v_ref[0].astype(jnp.bfloat16),
            preferred_element_type=jnp.float32)
        m_scr[...] = m_new

    @pl.when(kj == qi)
    def _():
        o_ref[0] = acc_scr[...] * (1.0 / l_scr[...])


def _know_kernel(attn_ref, x_ref, eo_ref, ln2s_ref, ln2b_ref,
                 wkcat_ref, bkcat_ref, kembT_ref, k_enc_ref, k_dec_ref,
                 y_out, uk_out, *, kknow):
    i = pl.program_id(0)
    x2 = x_ref[...] + jnp.dot(attn_ref[...].astype(jnp.bfloat16),
                              eo_ref[...].astype(jnp.bfloat16),
                              preferred_element_type=jnp.float32)
    h2 = _layer_norm(x2, ln2s_ref[...], ln2b_ref[...])
    hp = jnp.dot(h2, wkcat_ref[...], preferred_element_type=jnp.float32)
    hp = hp + bkcat_ref[...]
    h_k, tau_k = hp[:, 0:64], hp[:, 64:65]

    kembT = kembT_ref[...]
    s_k = jnp.dot(h_k, kembT, preferred_element_type=jnp.float32) * _inv_norms(kembT)
    y_out[...] = x2 + _bottleneck(_gate_dense(s_k, tau_k, kknow),
                                  k_enc_ref[...], k_dec_ref[...])

    @pl.when(i == 0)
    def _():
        uk_out[...] = jnp.zeros_like(uk_out)

    uk_out[...] += _softmax_colsum(s_k)


def _aux_kernel(uq_ref, uv_ref, uk_ref, out_ref, *, n_rows):
    nqk = uq_ref.shape[-1]
    nv = uv_ref.shape[-1]
    nk = uk_ref.shape[-1]
    uq = uq_ref[...] / n_rows - 1.0 / nqk
    uv = uv_ref[...] / n_rows - 1.0 / nv
    uk = uk_ref[...] / n_rows - 1.0 / nk
    aux = (jnp.sum(uq * uq) * nqk * 3 + jnp.sum(uv * uv) * nv
           + jnp.sum(uk * uk) * nk)
    out_ref[...] = aux.reshape(1, 1)


def kernel(x, qk_emb, qk_w_enc, qk_w_dec, v_emb, v_w_enc, v_w_dec,
           know_emb, know_w_enc, know_w_dec, proj_attn_k, proj_attn_b,
           tau_attn_k, tau_attn_b, proj_know_k, proj_know_b, tau_know_k,
           tau_know_b, expand_O, ln1_scale, ln1_bias, ln2_scale, ln2_bias):
    B, S, D = x.shape
    nqk, demb = qk_emb.shape
    nv = v_emb.shape[0]
    nk = know_emb.shape[0]
    kqk = min(128, nqk)
    kv = min(128, nv)
    kknow = min(256, nk)
    x2d = x.reshape(S, D)

    wcat = jnp.concatenate(
        [proj_attn_k, tau_attn_k,
         jnp.zeros((D, 256 - 3 * demb - 3), jnp.float32)], axis=1)
    bcat = jnp.concatenate(
        [proj_attn_b, tau_attn_b, jnp.zeros((256 - 3 * demb - 3,), jnp.float32)]
    ).reshape(1, 256)
    wkcat = jnp.concatenate(
        [proj_know_k, tau_know_k,
         jnp.zeros((D, 128 - demb - 1), jnp.float32)], axis=1)
    bkcat = jnp.concatenate(
        [proj_know_b, tau_know_b, jnp.zeros((128 - demb - 1,), jnp.float32)]
    ).reshape(1, 128)
    qkT = qk_emb.T
    vT = v_emb.T
    kembT = know_emb.T
    ln1s, ln1b = ln1_scale.reshape(1, D), ln1_bias.reshape(1, D)
    ln2s, ln2b = ln2_scale.reshape(1, D), ln2_bias.reshape(1, D)

    row_spec = pl.BlockSpec((_TS, D), lambda i: (i, 0))
    full = lambda a: pl.BlockSpec(a.shape, lambda i: (0,) * a.ndim)

    q2d, k2d, v2d, uq, uv = pl.pallas_call(
        functools.partial(_attn_prep_kernel, kqk=kqk, kv=kv),
        grid=(S // _TS,),
        in_specs=[row_spec, full(ln1s), full(ln1b), full(wcat), full(bcat),
                  full(qkT), full(vT), full(qk_w_enc), full(qk_w_dec),
                  full(v_w_enc), full(v_w_dec)],
        out_specs=[row_spec, row_spec, row_spec,
                   pl.BlockSpec((1, nqk), lambda i: (0, 0)),
                   pl.BlockSpec((1, nv), lambda i: (0, 0))],
        out_shape=[jax.ShapeDtypeStruct((S, D), jnp.float32),
                   jax.ShapeDtypeStruct((S, D), jnp.float32),
                   jax.ShapeDtypeStruct((S, D), jnp.float32),
                   jax.ShapeDtypeStruct((1, nqk), jnp.float32),
                   jax.ShapeDtypeStruct((1, nv), jnp.float32)],
    )(x2d, ln1s, ln1b, wcat, bcat, qkT, vT, qk_w_enc, qk_w_dec,
      v_w_enc, v_w_dec)

    d_head = D // _NHEADS
    to_heads = lambda a: a.reshape(S, _NHEADS, d_head).transpose(1, 0, 2)
    q3, k3, v3 = to_heads(q2d), to_heads(k2d), to_heads(v2d)
    q_spec = pl.BlockSpec((1, _TQ, d_head), lambda h, qi, kj: (h, qi, 0))
    kv_spec = pl.BlockSpec((1, _TK, d_head),
                           lambda h, qi, kj: (h, jnp.minimum(kj, qi), 0))
    attn3 = pl.pallas_call(
        _attn_kernel,
        grid=(_NHEADS, S // _TQ, S // _TK),
        in_specs=[q_spec, kv_spec, kv_spec],
        out_specs=q_spec,
        out_shape=jax.ShapeDtypeStruct((_NHEADS, S, d_head), jnp.float32),
        scratch_shapes=[pltpu.VMEM((_TQ, 1), jnp.float32),
                        pltpu.VMEM((_TQ, 1), jnp.float32),
                        pltpu.VMEM((_TQ, d_head), jnp.float32)],
    )(q3, k3, v3)
    attn = attn3.transpose(1, 0, 2).reshape(S, D)

    y, uk = pl.pallas_call(
        functools.partial(_know_kernel, kknow=kknow),
        grid=(S // _TS,),
        in_specs=[row_spec, row_spec, full(expand_O), full(ln2s), full(ln2b),
                  full(wkcat), full(bkcat), full(kembT), full(know_w_enc),
                  full(know_w_dec)],
        out_specs=[row_spec, pl.BlockSpec((1, nk), lambda i: (0, 0))],
        out_shape=[jax.ShapeDtypeStruct((S, D), jnp.float32),
                   jax.ShapeDtypeStruct((1, nk), jnp.float32)],
    )(attn, x2d, expand_O, ln2s, ln2b, wkcat, bkcat, kembT,
      know_w_enc, know_w_dec)

    aux = pl.pallas_call(
        functools.partial(_aux_kernel, n_rows=float(B * S)),
        grid=(1,),
        in_specs=[full(uq), full(uv), full(uk)],
        out_specs=pl.BlockSpec((1, 1), lambda i: (0, 0)),
        out_shape=jax.ShapeDtypeStruct((1, 1), jnp.float32),
    )(uq, uv, uk)

    return y.reshape(B, S, D), aux.reshape(())


# speedup vs baseline: 1.0717x; 1.0717x over previous
import functools

import jax
import jax.numpy as jnp
from jax.experimental import pallas as pl
from jax.experimental.pallas import tpu as pltpu

_NHEADS = 16
_TS = 256
_TQ = 512
_TK = 512

_INT_MIN_VAL = -2147483648


def _layer_norm(x, scale, bias, eps=1e-06):
    mean = jnp.mean(x, axis=-1, keepdims=True)
    var = jnp.mean(jnp.square(x - mean), axis=-1, keepdims=True)
    return (x - mean) / jnp.sqrt(var + eps) * scale + bias


def _softmax_colsum(s):
    m = jnp.max(s, axis=-1, keepdims=True)
    e = jnp.exp(s - m)
    p = e * (1.0 / jnp.sum(e, axis=-1, keepdims=True))
    return jnp.sum(p, axis=0, keepdims=True)


def _gate_dense(scores, tau, k):
    int_min = jnp.int32(_INT_MIN_VAL)
    i32 = jax.lax.bitcast_convert_type(scores, jnp.int32)
    u = jnp.where(i32 < 0, ~i32, i32 | int_min)
    su = u ^ int_min

    rows = scores.shape[0]
    lo0 = jnp.min(su, axis=-1, keepdims=True)
    hi0 = jnp.max(su, axis=-1, keepdims=True)
    gap0 = hi0 - lo0
    done0 = ((gap0 == 0) | (gap0 == 1)).astype(jnp.int32)
    thr0 = lo0
    kf = jnp.float32(k)

    def cond(c):
        it, ndone = c[0], c[1]
        return jnp.logical_and(it < 80, ndone < rows)

    def body(c):
        it, ndone, lo, hi, thr, done, clo, chi = c
        hgap = jax.lax.shift_right_logical(hi - lo, 1)
        qgap = jnp.maximum(jax.lax.shift_right_logical(hi - lo, 2), 1)
        frac = jnp.clip((clo - kf) / jnp.maximum(clo - chi, 1.0), 0.25, 0.75)
        off = (hgap.astype(jnp.float32) * (2.0 * frac)).astype(jnp.int32)
        mid = jnp.clip(lo + off, lo + qgap, hi - qgap)
        cnt = jnp.sum((su >= mid).astype(jnp.int32), axis=-1, keepdims=True)
        cntf = cnt.astype(jnp.float32)
        ge = cnt >= k
        lo2 = jnp.where(ge, mid, lo)
        hi2 = jnp.where(ge, hi, mid)
        clo2 = jnp.where(ge, cntf, clo)
        chi2 = jnp.where(ge, chi, cntf)
        hit = cnt == k
        gap = hi2 - lo2
        collapse = (gap == 0) | (gap == 1)
        fresh = done == 0
        thr2 = jnp.where(fresh, jnp.where(hit, mid, lo2), thr)
        done2 = jnp.where(fresh & (hit | collapse), 1, done)
        return (it + 1, jnp.sum(done2),
                jnp.where(fresh, lo2, lo), jnp.where(fresh, hi2, hi),
                thr2, done2,
                jnp.where(fresh, clo2, clo), jnp.where(fresh, chi2, chi))

    init = (jnp.int32(0), jnp.sum(done0), lo0, hi0, thr0, done0,
            jnp.full((rows, 1), float(scores.shape[-1]), jnp.float32),
            jnp.ones((rows, 1), jnp.float32))
    thr = jax.lax.while_loop(cond, body, init)[4]
    mask = su >= thr

    raw = scores - tau
    meg = jnp.where(mask & (raw > 0), jnp.exp(raw) - 1.0, 0.0)
    gate_sum = jnp.sum(meg, axis=-1, keepdims=True) + 1e-08
    strength = jnp.tanh(jnp.max(meg, axis=-1, keepdims=True))
    return meg * (strength / gate_sum)


def _bottleneck(gate, enc, dec):
    h = jnp.dot(gate.astype(jnp.bfloat16), enc.astype(jnp.bfloat16),
                preferred_element_type=jnp.float32)
    return jnp.dot(h.astype(jnp.bfloat16), dec.astype(jnp.bfloat16),
                   preferred_element_type=jnp.float32)


def _inv_norms(embT):
    return 1.0 / (jnp.sqrt(jnp.sum(embT * embT, axis=0, keepdims=True)) + 1e-08)


def _attn_prep_kernel(x_ref, ln1s_ref, ln1b_ref, wcat_ref, bcat_ref,
                      qkT_ref, vT_ref, qk_enc_ref, qk_dec_ref,
                      v_enc_ref, v_dec_ref,
                      q_out, k_out, v_out, uq_out, uv_out, *, kqk, kv):
    i = pl.program_id(0)
    h1 = _layer_norm(x_ref[...], ln1s_ref[...], ln1b_ref[...])
    hp = jnp.dot(h1, wcat_ref[...], preferred_element_type=jnp.float32)
    hp = hp + bcat_ref[...]
    h_q, h_k, h_v = hp[:, 0:64], hp[:, 64:128], hp[:, 128:192]
    tau_q, tau_k, tau_v = hp[:, 192:193], hp[:, 193:194], hp[:, 194:195]

    qkT = qkT_ref[...]
    inv_qk = _inv_norms(qkT)
    s_q = jnp.dot(h_q, qkT, preferred_element_type=jnp.float32) * inv_qk
    s_k = jnp.dot(h_k, qkT, preferred_element_type=jnp.float32) * inv_qk
    vT = vT_ref[...]
    s_v = jnp.dot(h_v, vT, preferred_element_type=jnp.float32) * _inv_norms(vT)

    ts = s_q.shape[0]
    s_all = jnp.concatenate([s_q, s_k, s_v], axis=0)
    tau_all = jnp.concatenate([tau_q, tau_k, tau_v], axis=0)
    g_all = _gate_dense(s_all, tau_all, kqk)
    qk_enc, qk_dec = qk_enc_ref[...], qk_dec_ref[...]
    qk_both = _bottleneck(g_all[0:2 * ts], qk_enc, qk_dec)
    q_out[...] = qk_both[0:ts]
    k_out[...] = qk_both[ts:2 * ts]
    v_out[...] = _bottleneck(g_all[2 * ts:3 * ts],
                             v_enc_ref[...], v_dec_ref[...])

    @pl.when(i == 0)
    def _():
        uq_out[...] = jnp.zeros_like(uq_out)
        uv_out[...] = jnp.zeros_like(uv_out)

    uq_out[...] += _softmax_colsum(s_q)
    uv_out[...] += _softmax_colsum(s_v)


def _attn_kernel(q_ref, k_ref, v_ref, o_ref):
    qi = pl.program_id(1)
    q = (q_ref[0] * 0.125).astype(jnp.bfloat16)
    s = jax.lax.dot_general(q, k_ref[0].astype(jnp.bfloat16),
                            (((1,), (1,)), ((), ())),
                            preferred_element_type=jnp.float32)
    row = qi * _TQ + jax.lax.broadcasted_iota(jnp.int32, s.shape, 0)
    col = jax.lax.broadcasted_iota(jnp.int32, s.shape, 1)
    s = jnp.where(col <= row, s, jnp.finfo(jnp.float32).min)
    m = jnp.max(s, axis=-1, keepdims=True)
    e = jnp.exp(s - m)
    p = (e * (1.0 / jnp.sum(e, axis=-1, keepdims=True))).astype(jnp.bfloat16)
    o_ref[0] = jnp.dot(p, v_ref[0].astype(jnp.bfloat16),
                       preferred_element_type=jnp.float32)


def _know_kernel(attn_ref, x_ref, eo_ref, ln2s_ref, ln2b_ref,
                 wkcat_ref, bkcat_ref, kembT_ref, k_enc_ref, k_dec_ref,
                 y_out, uk_out, *, kknow):
    i = pl.program_id(0)
    x2 = x_ref[...] + jnp.dot(attn_ref[...].astype(jnp.bfloat16),
                              eo_ref[...].astype(jnp.bfloat16),
                              preferred_element_type=jnp.float32)
    h2 = _layer_norm(x2, ln2s_ref[...], ln2b_ref[...])
    hp = jnp.dot(h2, wkcat_ref[...], preferred_element_type=jnp.float32)
    hp = hp + bkcat_ref[...]
    h_k, tau_k = hp[:, 0:64], hp[:, 64:65]

    kembT = kembT_ref[...]
    s_k = jnp.dot(h_k, kembT, preferred_element_type=jnp.float32) * _inv_norms(kembT)
    y_out[...] = x2 + _bottleneck(_gate_dense(s_k, tau_k, kknow),
                                  k_enc_ref[...], k_dec_ref[...])

    @pl.when(i == 0)
    def _():
        uk_out[...] = jnp.zeros_like(uk_out)

    uk_out[...] += _softmax_colsum(s_k)


def _aux_kernel(uq_ref, uv_ref, uk_ref, out_ref, *, n_rows):
    nqk = uq_ref.shape[-1]
    nv = uv_ref.shape[-1]
    nk = uk_ref.shape[-1]
    uq = uq_ref[...] / n_rows - 1.0 / nqk
    uv = uv_ref[...] / n_rows - 1.0 / nv
    uk = uk_ref[...] / n_rows - 1.0 / nk
    aux = (jnp.sum(uq * uq) * nqk * 3 + jnp.sum(uv * uv) * nv
           + jnp.sum(uk * uk) * nk)
    out_ref[...] = aux.reshape(1, 1)


def kernel(x, qk_emb, qk_w_enc, qk_w_dec, v_emb, v_w_enc, v_w_dec,
           know_emb, know_w_enc, know_w_dec, proj_attn_k, proj_attn_b,
           tau_attn_k, tau_attn_b, proj_know_k, proj_know_b, tau_know_k,
           tau_know_b, expand_O, ln1_scale, ln1_bias, ln2_scale, ln2_bias):
    B, S, D = x.shape
    nqk, demb = qk_emb.shape
    nv = v_emb.shape[0]
    nk = know_emb.shape[0]
    kqk = min(128, nqk)
    kv = min(128, nv)
    kknow = min(256, nk)
    x2d = x.reshape(S, D)

    wcat = jnp.concatenate(
        [proj_attn_k, tau_attn_k,
         jnp.zeros((D, 256 - 3 * demb - 3), jnp.float32)], axis=1)
    bcat = jnp.concatenate(
        [proj_attn_b, tau_attn_b, jnp.zeros((256 - 3 * demb - 3,), jnp.float32)]
    ).reshape(1, 256)
    wkcat = jnp.concatenate(
        [proj_know_k, tau_know_k,
         jnp.zeros((D, 128 - demb - 1), jnp.float32)], axis=1)
    bkcat = jnp.concatenate(
        [proj_know_b, tau_know_b, jnp.zeros((128 - demb - 1,), jnp.float32)]
    ).reshape(1, 128)
    qkT = qk_emb.T
    vT = v_emb.T
    kembT = know_emb.T
    ln1s, ln1b = ln1_scale.reshape(1, D), ln1_bias.reshape(1, D)
    ln2s, ln2b = ln2_scale.reshape(1, D), ln2_bias.reshape(1, D)

    row_spec = pl.BlockSpec((_TS, D), lambda i: (i, 0))
    full = lambda a: pl.BlockSpec(a.shape, lambda i: (0,) * a.ndim)

    q2d, k2d, v2d, uq, uv = pl.pallas_call(
        functools.partial(_attn_prep_kernel, kqk=kqk, kv=kv),
        grid=(S // _TS,),
        in_specs=[row_spec, full(ln1s), full(ln1b), full(wcat), full(bcat),
                  full(qkT), full(vT), full(qk_w_enc), full(qk_w_dec),
                  full(v_w_enc), full(v_w_dec)],
        out_specs=[row_spec, row_spec, row_spec,
                   pl.BlockSpec((1, nqk), lambda i: (0, 0)),
                   pl.BlockSpec((1, nv), lambda i: (0, 0))],
        out_shape=[jax.ShapeDtypeStruct((S, D), jnp.float32),
                   jax.ShapeDtypeStruct((S, D), jnp.float32),
                   jax.ShapeDtypeStruct((S, D), jnp.float32),
                   jax.ShapeDtypeStruct((1, nqk), jnp.float32),
                   jax.ShapeDtypeStruct((1, nv), jnp.float32)],
    )(x2d, ln1s, ln1b, wcat, bcat, qkT, vT, qk_w_enc, qk_w_dec,
      v_w_enc, v_w_dec)

    d_head = D // _NHEADS
    to_heads = lambda a: a.reshape(S, _NHEADS, d_head).transpose(1, 0, 2)
    q3, k3, v3 = to_heads(q2d), to_heads(k2d), to_heads(v2d)
    q_spec = pl.BlockSpec((1, _TQ, d_head), lambda h, qi: (h, qi, 0))
    kv_spec = pl.BlockSpec((1, S, d_head), lambda h, qi: (h, 0, 0))
    attn3 = pl.pallas_call(
        _attn_kernel,
        grid=(_NHEADS, S // _TQ),
        in_specs=[q_spec, kv_spec, kv_spec],
        out_specs=q_spec,
        out_shape=jax.ShapeDtypeStruct((_NHEADS, S, d_head), jnp.float32),
    )(q3, k3, v3)
    attn = attn3.transpose(1, 0, 2).reshape(S, D)

    y, uk = pl.pallas_call(
        functools.partial(_know_kernel, kknow=kknow),
        grid=(S // _TS,),
        in_specs=[row_spec, row_spec, full(expand_O), full(ln2s), full(ln2b),
                  full(wkcat), full(bkcat), full(kembT), full(know_w_enc),
                  full(know_w_dec)],
        out_specs=[row_spec, pl.BlockSpec((1, nk), lambda i: (0, 0))],
        out_shape=[jax.ShapeDtypeStruct((S, D), jnp.float32),
                   jax.ShapeDtypeStruct((1, nk), jnp.float32)],
    )(attn, x2d, expand_O, ln2s, ln2b, wkcat, bkcat, kembT,
      know_w_enc, know_w_dec)

    aux = pl.pallas_call(
        functools.partial(_aux_kernel, n_rows=float(B * S)),
        grid=(1,),
        in_specs=[full(uq), full(uv), full(uk)],
        out_specs=pl.BlockSpec((1, 1), lambda i: (0, 0)),
        out_shape=jax.ShapeDtypeStruct((1, 1), jnp.float32),
    )(uq, uv, uk)

    return y.reshape(B, S, D), aux.reshape(())


# speedup vs baseline: 1.1262x; 1.0508x over previous
import functools

import jax
import jax.numpy as jnp
from jax.experimental import pallas as pl
from jax.experimental.pallas import tpu as pltpu

_NHEADS = 16
_TS = 256
_TQ = 256

_INT_MIN_VAL = -2147483648


def _layer_norm(x, scale, bias, eps=1e-06):
    mean = jnp.mean(x, axis=-1, keepdims=True)
    var = jnp.mean(jnp.square(x - mean), axis=-1, keepdims=True)
    return (x - mean) / jnp.sqrt(var + eps) * scale + bias


def _softmax_colsum(s):
    m = jnp.max(s, axis=-1, keepdims=True)
    e = jnp.exp(s - m)
    p = e * (1.0 / jnp.sum(e, axis=-1, keepdims=True))
    return jnp.sum(p, axis=0, keepdims=True)


def _gate_dense(scores, tau, k):
    i32 = jax.lax.bitcast_convert_type(scores, jnp.int32)
    su = jnp.where(i32 < 0, i32 ^ jnp.int32(0x7FFFFFFF), i32)

    rows = scores.shape[0]
    lo0 = jnp.min(su, axis=-1, keepdims=True)
    hi0 = jnp.max(su, axis=-1, keepdims=True)
    gap0 = hi0 - lo0
    done0 = ((gap0 == 0) | (gap0 == 1)).astype(jnp.int32)
    thr0 = lo0
    kf = jnp.float32(k)

    def cond(c):
        it, ndone = c[0], c[1]
        return jnp.logical_and(it < 80, ndone < rows)

    def body(c):
        it, ndone, lo, hi, thr, done, clo, chi = c
        hgap = jax.lax.shift_right_logical(hi - lo, 1)
        qgap = jnp.maximum(jax.lax.shift_right_logical(hi - lo, 2), 1)
        frac = jnp.clip((clo - kf) / jnp.maximum(clo - chi, 1.0), 0.25, 0.75)
        off = (hgap.astype(jnp.float32) * (2.0 * frac)).astype(jnp.int32)
        mid = jnp.clip(lo + off, lo + qgap, hi - qgap)
        cnt = jnp.sum((su >= mid).astype(jnp.int32), axis=-1, keepdims=True)
        cntf = cnt.astype(jnp.float32)
        ge = cnt >= k
        lo2 = jnp.where(ge, mid, lo)
        hi2 = jnp.where(ge, hi, mid)
        clo2 = jnp.where(ge, cntf, clo)
        chi2 = jnp.where(ge, chi, cntf)
        hit = cnt == k
        gap = hi2 - lo2
        collapse = (gap == 0) | (gap == 1)
        fresh = done == 0
        thr2 = jnp.where(fresh, jnp.where(hit, mid, lo2), thr)
        done2 = jnp.where(fresh & (hit | collapse), 1, done)
        return (it + 1, jnp.sum(done2),
                jnp.where(fresh, lo2, lo), jnp.where(fresh, hi2, hi),
                thr2, done2,
                jnp.where(fresh, clo2, clo), jnp.where(fresh, chi2, chi))

    init = (jnp.int32(0), jnp.sum(done0), lo0, hi0, thr0, done0,
            jnp.full((rows, 1), float(scores.shape[-1]), jnp.float32),
            jnp.ones((rows, 1), jnp.float32))
    thr = jax.lax.while_loop(cond, body, init)[4]
    mask = su >= thr

    raw = scores - tau
    meg = jnp.where(mask & (raw > 0), jnp.exp(raw) - 1.0, 0.0)
    gate_sum = jnp.sum(meg, axis=-1, keepdims=True) + 1e-08
    strength = jnp.tanh(jnp.max(meg, axis=-1, keepdims=True))
    return meg * (strength / gate_sum)


def _bottleneck(gate, enc, dec):
    h = jnp.dot(gate.astype(jnp.bfloat16), enc.astype(jnp.bfloat16),
                preferred_element_type=jnp.float32)
    return jnp.dot(h.astype(jnp.bfloat16), dec.astype(jnp.bfloat16),
                   preferred_element_type=jnp.float32)


def _inv_norms(embT):
    return 1.0 / (jnp.sqrt(jnp.sum(embT * embT, axis=0, keepdims=True)) + 1e-08)


def _attn_prep_kernel(x_ref, ln1s_ref, ln1b_ref, wcat_ref, bcat_ref,
                      qkT_ref, vT_ref, qk_enc_ref, qk_dec_ref,
                      v_enc_ref, v_dec_ref,
                      q_out, k_out, v_out, uq_out, uv_out, *, kqk, kv):
    i = pl.program_id(0)
    h1 = _layer_norm(x_ref[...], ln1s_ref[...], ln1b_ref[...])
    hp = jnp.dot(h1, wcat_ref[...], preferred_element_type=jnp.float32)
    hp = hp + bcat_ref[...]
    h_q, h_k, h_v = hp[:, 0:64], hp[:, 64:128], hp[:, 128:192]
    tau_q, tau_k, tau_v = hp[:, 192:193], hp[:, 193:194], hp[:, 194:195]

    qkT = qkT_ref[...]
    inv_qk = _inv_norms(qkT)
    s_q = jnp.dot(h_q, qkT, preferred_element_type=jnp.float32) * inv_qk
    s_k = jnp.dot(h_k, qkT, preferred_element_type=jnp.float32) * inv_qk
    vT = vT_ref[...]
    s_v = jnp.dot(h_v, vT, preferred_element_type=jnp.float32) * _inv_norms(vT)

    ts = s_q.shape[0]
    s_all = jnp.concatenate([s_q, s_k, s_v], axis=0)
    tau_all = jnp.concatenate([tau_q, tau_k, tau_v], axis=0)
    g_all = _gate_dense(s_all, tau_all, kqk)
    qk_enc, qk_dec = qk_enc_ref[...], qk_dec_ref[...]
    qk_both = _bottleneck(g_all[0:2 * ts], qk_enc, qk_dec)
    q_out[...] = qk_both[0:ts]
    k_out[...] = qk_both[ts:2 * ts]
    v_out[...] = _bottleneck(g_all[2 * ts:3 * ts],
                             v_enc_ref[...], v_dec_ref[...])

    @pl.when(i == 0)
    def _():
        uq_out[...] = jnp.zeros_like(uq_out)
        uv_out[...] = jnp.zeros_like(uv_out)

    uq_out[...] += _softmax_colsum(s_q)
    uv_out[...] += _softmax_colsum(s_v)


def _attn_kernel(q_ref, k_ref, v_ref, o_ref):
    qi = pl.program_id(1)
    q = (q_ref[0] * 0.125).astype(jnp.bfloat16)
    s = jax.lax.dot_general(q, k_ref[0].astype(jnp.bfloat16),
                            (((1,), (1,)), ((), ())),
                            preferred_element_type=jnp.float32)
    row = qi * _TQ + jax.lax.broadcasted_iota(jnp.int32, s.shape, 0)
    col = jax.lax.broadcasted_iota(jnp.int32, s.shape, 1)
    s = jnp.where(col <= row, s, jnp.finfo(jnp.float32).min)
    m = jnp.max(s, axis=-1, keepdims=True)
    e = jnp.exp(s - m)
    p = (e * (1.0 / jnp.sum(e, axis=-1, keepdims=True))).astype(jnp.bfloat16)
    o_ref[0] = jnp.dot(p, v_ref[0].astype(jnp.bfloat16),
                       preferred_element_type=jnp.float32)


def _know_kernel(attn_ref, x_ref, eo_ref, ln2s_ref, ln2b_ref,
                 wkcat_ref, bkcat_ref, kembT_ref, k_enc_ref, k_dec_ref,
                 uq_ref, uv_ref, y_out, uk_out, aux_out, *, kknow, n_rows):
    i = pl.program_id(0)
    x2 = x_ref[...] + jnp.dot(attn_ref[...].astype(jnp.bfloat16),
                              eo_ref[...].astype(jnp.bfloat16),
                              preferred_element_type=jnp.float32)
    h2 = _layer_norm(x2, ln2s_ref[...], ln2b_ref[...])
    hp = jnp.dot(h2, wkcat_ref[...], preferred_element_type=jnp.float32)
    hp = hp + bkcat_ref[...]
    h_k, tau_k = hp[:, 0:64], hp[:, 64:65]

    kembT = kembT_ref[...]
    s_k = jnp.dot(h_k, kembT, preferred_element_type=jnp.float32) * _inv_norms(kembT)
    y_out[...] = x2 + _bottleneck(_gate_dense(s_k, tau_k, kknow),
                                  k_enc_ref[...], k_dec_ref[...])

    @pl.when(i == 0)
    def _():
        uk_out[...] = jnp.zeros_like(uk_out)

    uk_out[...] += _softmax_colsum(s_k)

    @pl.when(i == pl.num_programs(0) - 1)
    def _():
        nqk = uq_ref.shape[-1]
        nv = uv_ref.shape[-1]
        nk = uk_out.shape[-1]
        uq = uq_ref[...] / n_rows - 1.0 / nqk
        uv = uv_ref[...] / n_rows - 1.0 / nv
        uk = uk_out[...] / n_rows - 1.0 / nk
        aux = (jnp.sum(uq * uq) * nqk * 3 + jnp.sum(uv * uv) * nv
               + jnp.sum(uk * uk) * nk)
        aux_out[...] = aux.reshape(1, 1)


def kernel(x, qk_emb, qk_w_enc, qk_w_dec, v_emb, v_w_enc, v_w_dec,
           know_emb, know_w_enc, know_w_dec, proj_attn_k, proj_attn_b,
           tau_attn_k, tau_attn_b, proj_know_k, proj_know_b, tau_know_k,
           tau_know_b, expand_O, ln1_scale, ln1_bias, ln2_scale, ln2_bias):
    B, S, D = x.shape
    nqk, demb = qk_emb.shape
    nv = v_emb.shape[0]
    nk = know_emb.shape[0]
    kqk = min(128, nqk)
    kv = min(128, nv)
    kknow = min(256, nk)
    x2d = x.reshape(S, D)

    wcat = jnp.concatenate(
        [proj_attn_k, tau_attn_k,
         jnp.zeros((D, 256 - 3 * demb - 3), jnp.float32)], axis=1)
    bcat = jnp.concatenate(
        [proj_attn_b, tau_attn_b, jnp.zeros((256 - 3 * demb - 3,), jnp.float32)]
    ).reshape(1, 256)
    wkcat = jnp.concatenate(
        [proj_know_k, tau_know_k,
         jnp.zeros((D, 128 - demb - 1), jnp.float32)], axis=1)
    bkcat = jnp.concatenate(
        [proj_know_b, tau_know_b, jnp.zeros((128 - demb - 1,), jnp.float32)]
    ).reshape(1, 128)
    qkT = qk_emb.T
    vT = v_emb.T
    kembT = know_emb.T
    ln1s, ln1b = ln1_scale.reshape(1, D), ln1_bias.reshape(1, D)
    ln2s, ln2b = ln2_scale.reshape(1, D), ln2_bias.reshape(1, D)

    row_spec = pl.BlockSpec((_TS, D), lambda i: (i, 0))
    full = lambda a: pl.BlockSpec(a.shape, lambda i: (0,) * a.ndim)

    q2d, k2d, v2d, uq, uv = pl.pallas_call(
        functools.partial(_attn_prep_kernel, kqk=kqk, kv=kv),
        grid=(S // _TS,),
        in_specs=[row_spec, full(ln1s), full(ln1b), full(wcat), full(bcat),
                  full(qkT), full(vT), full(qk_w_enc), full(qk_w_dec),
                  full(v_w_enc), full(v_w_dec)],
        out_specs=[row_spec, row_spec, row_spec,
                   pl.BlockSpec((1, nqk), lambda i: (0, 0)),
                   pl.BlockSpec((1, nv), lambda i: (0, 0))],
        out_shape=[jax.ShapeDtypeStruct((S, D), jnp.float32),
                   jax.ShapeDtypeStruct((S, D), jnp.float32),
                   jax.ShapeDtypeStruct((S, D), jnp.float32),
                   jax.ShapeDtypeStruct((1, nqk), jnp.float32),
                   jax.ShapeDtypeStruct((1, nv), jnp.float32)],
    )(x2d, ln1s, ln1b, wcat, bcat, qkT, vT, qk_w_enc, qk_w_dec,
      v_w_enc, v_w_dec)

    d_head = D // _NHEADS
    to_heads = lambda a: a.reshape(S, _NHEADS, d_head).transpose(1, 0, 2)
    q3, k3, v3 = to_heads(q2d), to_heads(k2d), to_heads(v2d)
    q_spec = pl.BlockSpec((1, _TQ, d_head), lambda h, qi: (h, qi, 0))
    kv_spec = pl.BlockSpec((1, S, d_head), lambda h, qi: (h, 0, 0))
    attn3 = pl.pallas_call(
        _attn_kernel,
        grid=(_NHEADS, S // _TQ),
        in_specs=[q_spec, kv_spec, kv_spec],
        out_specs=q_spec,
        out_shape=jax.ShapeDtypeStruct((_NHEADS, S, d_head), jnp.float32),
    )(q3, k3, v3)
    attn = attn3.transpose(1, 0, 2).reshape(S, D)

    y, _, aux = pl.pallas_call(
        functools.partial(_know_kernel, kknow=kknow, n_rows=float(B * S)),
        grid=(S // _TS,),
        in_specs=[row_spec, row_spec, full(expand_O), full(ln2s), full(ln2b),
                  full(wkcat), full(bkcat), full(kembT), full(know_w_enc),
                  full(know_w_dec), full(uq), full(uv)],
        out_specs=[row_spec, pl.BlockSpec((1, nk), lambda i: (0, 0)),
                   pl.BlockSpec((1, 1), lambda i: (0, 0))],
        out_shape=[jax.ShapeDtypeStruct((S, D), jnp.float32),
                   jax.ShapeDtypeStruct((1, nk), jnp.float32),
                   jax.ShapeDtypeStruct((1, 1), jnp.float32)],
    )(attn, x2d, expand_O, ln2s, ln2b, wkcat, bkcat, kembT,
      know_w_enc, know_w_dec, uq, uv)

    return y.reshape(B, S, D), aux.reshape(())


# speedup vs baseline: 1.1277x; 1.0013x over previous
import functools

import jax
import jax.numpy as jnp
from jax.experimental import pallas as pl

_NHEADS = 16
_TS = 256
_TQ = 256


def _layer_norm(x, scale, bias, eps=1e-06):
    mean = jnp.mean(x, axis=-1, keepdims=True)
    var = jnp.mean(jnp.square(x - mean), axis=-1, keepdims=True)
    return (x - mean) / jnp.sqrt(var + eps) * scale + bias


def _softmax_colsum(s):
    m = jnp.max(s, axis=-1, keepdims=True)
    e = jnp.exp(s - m)
    p = e * (1.0 / jnp.sum(e, axis=-1, keepdims=True))
    return jnp.sum(p, axis=0, keepdims=True)


def _gate_dense(scores, tau, k):
    i32 = jax.lax.bitcast_convert_type(scores, jnp.int32)
    su = jnp.where(i32 < 0, i32 ^ jnp.int32(0x7FFFFFFF), i32)

    rows = scores.shape[0]
    lo0 = jnp.min(su, axis=-1, keepdims=True)
    hi0 = jnp.max(su, axis=-1, keepdims=True)
    gap0 = hi0 - lo0
    done0 = ((gap0 == 0) | (gap0 == 1)).astype(jnp.int32)
    thr0 = lo0
    kf = jnp.float32(k)

    def cond(c):
        it, ndone = c[0], c[1]
        return jnp.logical_and(it < 80, ndone < rows)

    def body(c):
        it, ndone, lo, hi, thr, done, clo, chi = c
        hgap = jax.lax.shift_right_logical(hi - lo, 1)
        qgap = jnp.maximum(jax.lax.shift_right_logical(hi - lo, 2), 1)
        frac = jnp.clip((clo - kf) / jnp.maximum(clo - chi, 1.0), 0.25, 0.75)
        off = (hgap.astype(jnp.float32) * (2.0 * frac)).astype(jnp.int32)
        mid = jnp.clip(lo + off, lo + qgap, hi - qgap)
        cnt = jnp.sum((su >= mid).astype(jnp.int32), axis=-1, keepdims=True)
        cntf = cnt.astype(jnp.float32)
        ge = cnt >= k
        lo2 = jnp.where(ge, mid, lo)
        hi2 = jnp.where(ge, hi, mid)
        clo2 = jnp.where(ge, cntf, clo)
        chi2 = jnp.where(ge, chi, cntf)
        hit = cnt == k
        gap = hi2 - lo2
        collapse = (gap == 0) | (gap == 1)
        fresh = done == 0
        thr2 = jnp.where(fresh, jnp.where(hit, mid, lo2), thr)
        done2 = jnp.where(fresh & (hit | collapse), 1, done)
        return (it + 1, jnp.sum(done2),
                jnp.where(fresh, lo2, lo), jnp.where(fresh, hi2, hi),
                thr2, done2,
                jnp.where(fresh, clo2, clo), jnp.where(fresh, chi2, chi))

    init = (jnp.int32(0), jnp.sum(done0), lo0, hi0, thr0, done0,
            jnp.full((rows, 1), float(scores.shape[-1]), jnp.float32),
            jnp.ones((rows, 1), jnp.float32))
    thr = jax.lax.while_loop(cond, body, init)[4]
    mask = su >= thr

    raw = scores - tau
    meg = jnp.where(mask & (raw > 0), jnp.exp(raw) - 1.0, 0.0)
    gate_sum = jnp.sum(meg, axis=-1, keepdims=True) + 1e-08
    strength = jnp.tanh(jnp.max(meg, axis=-1, keepdims=True))
    return meg * (strength / gate_sum)


def _bottleneck(gate, enc, dec):
    h = jnp.dot(gate.astype(jnp.bfloat16), enc.astype(jnp.bfloat16),
                preferred_element_type=jnp.float32)
    return jnp.dot(h.astype(jnp.bfloat16), dec.astype(jnp.bfloat16),
                   preferred_element_type=jnp.float32)


def _inv_norms(embT):
    return 1.0 / (jnp.sqrt(jnp.sum(embT * embT, axis=0, keepdims=True)) + 1e-08)


def _attn_prep_kernel(x_ref, ln1s_ref, ln1b_ref, wcat_ref, bcat_ref,
                      qkT_ref, vT_ref, qk_enc_ref, qk_dec_ref,
                      v_enc_ref, v_dec_ref,
                      q_out, k_out, v_out, uq_out, uv_out, *, kqk, kv):
    i = pl.program_id(0)
    h1 = _layer_norm(x_ref[...], ln1s_ref[...], ln1b_ref[...])
    hp = jnp.dot(h1, wcat_ref[...], preferred_element_type=jnp.float32)
    hp = hp + bcat_ref[...]
    h_q, h_k, h_v = hp[:, 0:64], hp[:, 64:128], hp[:, 128:192]
    tau_q, tau_k, tau_v = hp[:, 192:193], hp[:, 193:194], hp[:, 194:195]

    qkT = qkT_ref[...]
    inv_qk = _inv_norms(qkT)
    s_q = jnp.dot(h_q, qkT, preferred_element_type=jnp.float32) * inv_qk
    s_k = jnp.dot(h_k, qkT, preferred_element_type=jnp.float32) * inv_qk
    vT = vT_ref[...]
    s_v = jnp.dot(h_v, vT, preferred_element_type=jnp.float32) * _inv_norms(vT)

    ts = s_q.shape[0]
    s_all = jnp.concatenate([s_q, s_k, s_v], axis=0)
    tau_all = jnp.concatenate([tau_q, tau_k, tau_v], axis=0)
    g_all = _gate_dense(s_all, tau_all, kqk)
    qk_enc, qk_dec = qk_enc_ref[...], qk_dec_ref[...]
    qk_both = _bottleneck(g_all[0:2 * ts], qk_enc, qk_dec)
    q_out[...] = qk_both[0:ts]
    k_out[...] = qk_both[ts:2 * ts]
    v_out[...] = _bottleneck(g_all[2 * ts:3 * ts],
                             v_enc_ref[...], v_dec_ref[...])

    @pl.when(i == 0)
    def _():
        uq_out[...] = jnp.zeros_like(uq_out)
        uv_out[...] = jnp.zeros_like(uv_out)

    uq_out[...] += _softmax_colsum(s_q)
    uv_out[...] += _softmax_colsum(s_v)


def _attn_kernel(q_ref, k_ref, v_ref, o_ref):
    qi = pl.program_id(1)
    q = (q_ref[0] * 0.125).astype(jnp.bfloat16)
    s = jax.lax.dot_general(q, k_ref[0].astype(jnp.bfloat16),
                            (((1,), (1,)), ((), ())),
                            preferred_element_type=jnp.float32)
    row = qi * _TQ + jax.lax.broadcasted_iota(jnp.int32, s.shape, 0)
    col = jax.lax.broadcasted_iota(jnp.int32, s.shape, 1)
    s = jnp.where(col <= row, s, jnp.finfo(jnp.float32).min)
    m = jnp.max(s, axis=-1, keepdims=True)
    e = jnp.exp(s - m)
    p = (e * (1.0 / jnp.sum(e, axis=-1, keepdims=True))).astype(jnp.bfloat16)
    o_ref[0] = jnp.dot(p, v_ref[0].astype(jnp.bfloat16),
                       preferred_element_type=jnp.float32)


def _know_kernel(attn_ref, x_ref, eo_ref, ln2s_ref, ln2b_ref,
                 wkcat_ref, bkcat_ref, kembT_ref, k_enc_ref, k_dec_ref,
                 uq_ref, uv_ref, y_out, uk_out, aux_out, *, kknow, n_rows):
    i = pl.program_id(0)
    x2 = x_ref[...] + jnp.dot(attn_ref[...].astype(jnp.bfloat16),
                              eo_ref[...].astype(jnp.bfloat16),
                              preferred_element_type=jnp.float32)
    h2 = _layer_norm(x2, ln2s_ref[...], ln2b_ref[...])
    hp = jnp.dot(h2, wkcat_ref[...], preferred_element_type=jnp.float32)
    hp = hp + bkcat_ref[...]
    h_k, tau_k = hp[:, 0:64], hp[:, 64:65]

    kembT = kembT_ref[...]
    s_k = jnp.dot(h_k, kembT, preferred_element_type=jnp.float32) * _inv_norms(kembT)
    y_out[...] = x2 + _bottleneck(_gate_dense(s_k, tau_k, kknow),
                                  k_enc_ref[...], k_dec_ref[...])

    @pl.when(i == 0)
    def _():
        uk_out[...] = jnp.zeros_like(uk_out)

    uk_out[...] += _softmax_colsum(s_k)

    @pl.when(i == pl.num_programs(0) - 1)
    def _():
        nqk = uq_ref.shape[-1]
        nv = uv_ref.shape[-1]
        nk = uk_out.shape[-1]
        uq = uq_ref[...] / n_rows - 1.0 / nqk
        uv = uv_ref[...] / n_rows - 1.0 / nv
        uk = uk_out[...] / n_rows - 1.0 / nk
        aux = (jnp.sum(uq * uq) * nqk * 3 + jnp.sum(uv * uv) * nv
               + jnp.sum(uk * uk) * nk)
        aux_out[...] = aux.reshape(1, 1)


def kernel(x, qk_emb, qk_w_enc, qk_w_dec, v_emb, v_w_enc, v_w_dec,
           know_emb, know_w_enc, know_w_dec, proj_attn_k, proj_attn_b,
           tau_attn_k, tau_attn_b, proj_know_k, proj_know_b, tau_know_k,
           tau_know_b, expand_O, ln1_scale, ln1_bias, ln2_scale, ln2_bias):
    B, S, D = x.shape
    nqk, demb = qk_emb.shape
    nv = v_emb.shape[0]
    nk = know_emb.shape[0]
    kqk = min(128, nqk)
    kv = min(128, nv)
    kknow = min(256, nk)
    assert kqk == kv
    x2d = x.reshape(S, D)

    wcat = jnp.concatenate(
        [proj_attn_k, tau_attn_k,
         jnp.zeros((D, 256 - 3 * demb - 3), jnp.float32)], axis=1)
    bcat = jnp.concatenate(
        [proj_attn_b, tau_attn_b, jnp.zeros((256 - 3 * demb - 3,), jnp.float32)]
    ).reshape(1, 256)
    wkcat = jnp.concatenate(
        [proj_know_k, tau_know_k,
         jnp.zeros((D, 128 - demb - 1), jnp.float32)], axis=1)
    bkcat = jnp.concatenate(
        [proj_know_b, tau_know_b, jnp.zeros((128 - demb - 1,), jnp.float32)]
    ).reshape(1, 128)
    qkT = qk_emb.T
    vT = v_emb.T
    kembT = know_emb.T
    ln1s, ln1b = ln1_scale.reshape(1, D), ln1_bias.reshape(1, D)
    ln2s, ln2b = ln2_scale.reshape(1, D), ln2_bias.reshape(1, D)

    row_spec = pl.BlockSpec((_TS, D), lambda i: (i, 0))
    full = lambda a: pl.BlockSpec(a.shape, lambda i: (0,) * a.ndim)

    q2d, k2d, v2d, uq, uv = pl.pallas_call(
        functools.partial(_attn_prep_kernel, kqk=kqk, kv=kv),
        grid=(S // _TS,),
        in_specs=[row_spec, full(ln1s), full(ln1b), full(wcat), full(bcat),
                  full(qkT), full(vT), full(qk_w_enc), full(qk_w_dec),
                  full(v_w_enc), full(v_w_dec)],
        out_specs=[row_spec, row_spec, row_spec,
                   pl.BlockSpec((1, nqk), lambda i: (0, 0)),
                   pl.BlockSpec((1, nv), lambda i: (0, 0))],
        out_shape=[jax.ShapeDtypeStruct((S, D), jnp.float32),
                   jax.ShapeDtypeStruct((S, D), jnp.float32),
                   jax.ShapeDtypeStruct((S, D), jnp.float32),
                   jax.ShapeDtypeStruct((1, nqk), jnp.float32),
                   jax.ShapeDtypeStruct((1, nv), jnp.float32)],
    )(x2d, ln1s, ln1b, wcat, bcat, qkT, vT, qk_w_enc, qk_w_dec,
      v_w_enc, v_w_dec)

    d_head = D // _NHEADS
    to_heads = lambda a: a.reshape(S, _NHEADS, d_head).transpose(1, 0, 2)
    q3, k3, v3 = to_heads(q2d), to_heads(k2d), to_heads(v2d)
    q_spec = pl.BlockSpec((1, _TQ, d_head), lambda h, qi: (h, qi, 0))
    kv_spec = pl.BlockSpec((1, S, d_head), lambda h, qi: (h, 0, 0))
    attn3 = pl.pallas_call(
        _attn_kernel,
        grid=(_NHEADS, S // _TQ),
        in_specs=[q_spec, kv_spec, kv_spec],
        out_specs=q_spec,
        out_shape=jax.ShapeDtypeStruct((_NHEADS, S, d_head), jnp.float32),
    )(q3, k3, v3)
    attn = attn3.transpose(1, 0, 2).reshape(S, D)

    y, _, aux = pl.pallas_call(
        functools.partial(_know_kernel, kknow=kknow, n_rows=float(B * S)),
        grid=(S // _TS,),
        in_specs=[row_spec, row_spec, full(expand_O), full(ln2s), full(ln2b),
                  full(wkcat), full(bkcat), full(kembT), full(know_w_enc),
                  full(know_w_dec), full(uq), full(uv)],
        out_specs=[row_spec, pl.BlockSpec((1, nk), lambda i: (0, 0)),
                   pl.BlockSpec((1, 1), lambda i: (0, 0))],
        out_shape=[jax.ShapeDtypeStruct((S, D), jnp.float32),
                   jax.ShapeDtypeStruct((1, nk), jnp.float32),
                   jax.ShapeDtypeStruct((1, 1), jnp.float32)],
    )(attn, x2d, expand_O, ln2s, ln2b, wkcat, bkcat, kembT,
      know_w_enc, know_w_dec, uq, uv)

    return y.reshape(B, S, D), aux.reshape(())
